# Initial kernel scaffold; baseline (speedup 1.0000x reference)
import functools

import jax
import jax.numpy as jnp
from jax import lax
from jax.experimental import pallas as pl
from jax.experimental.pallas import tpu as pltpu
from jax.experimental.pallas import tpu_sc as plsc

D = 64
NC, NS = 2, 16
NW = NC * NS
CH = 128
GPS = 4
SUPER = CH * GPS


@functools.cache
def _build(B):
    BPW = B // NW
    NCH = BPW // CH
    NSUP = BPW // SUPER
    assert B % (NW * SUPER) == 0 and NSUP % 2 == 0

    mesh = plsc.VectorSubcoreMesh(core_axis_name="c", subcore_axis_name="s")

    @functools.partial(
        pl.kernel,
        out_type=jax.ShapeDtypeStruct((B, D), jnp.float32),
        mesh=mesh,
        scratch_types=[
            pltpu.VMEM((NCH, CH), jnp.int32),
            pltpu.VMEM((SUPER, D), jnp.float32),
            pltpu.VMEM((SUPER, D), jnp.float32),
            pltpu.SemaphoreType.DMA,
            pltpu.SemaphoreType.DMA,
        ],
    )
    def emb(idx_hbm, table_hbm, out_hbm, idx_v, buf0, buf1, sem0, sem1):
        wid = lax.axis_index("s") * NC + lax.axis_index("c")
        base = wid * BPW
        pltpu.sync_copy(idx_hbm.at[pl.ds(wid * NCH, NCH)], idx_v)
        bufs = (buf0, buf1)
        sems = (sem0, sem1)

        def fire(g, b):
            for j in range(GPS):
                pltpu.make_async_copy(
                    table_hbm.at[idx_v.at[g * GPS + j]],
                    bufs[b].at[pl.ds(j * CH, CH)],
                    sems[b],
                ).start()

        def drain(g, b):
            for j in range(GPS):
                pltpu.make_async_copy(
                    table_hbm.at[idx_v.at[g * GPS + j]],
                    bufs[b].at[pl.ds(j * CH, CH)],
                    sems[b],
                ).wait()

        fire(0, 0)

        @pl.loop(0, NSUP, step=2)
        def _super(g0):
            for b in range(2):
                g = g0 + b

                @pl.when(g + 1 < NSUP)
                def _():
                    fire(g + 1, 1 - b)

                drain(g, b)
                pltpu.sync_copy(
                    bufs[b], out_hbm.at[pl.ds(base + g * SUPER, SUPER)]
                )

    return emb


def kernel(x, table):
    S, T = x.shape
    B = S * T
    idx = x.reshape(B // CH, CH).astype(jnp.int32)
    out = _build(B)(idx, table)
    return out.reshape(S, T, D)


# speedup vs baseline: 1.8738x; 1.8738x over previous
import functools

import jax
import jax.numpy as jnp
from jax import lax
from jax.experimental import pallas as pl
from jax.experimental.pallas import tpu as pltpu
from jax.experimental.pallas import tpu_sc as plsc

D = 64
NC, NS = 2, 16
NW = NC * NS
CH = 128
GPS = 4
SUPER = CH * GPS


@functools.cache
def _build(B):
    BPW = B // NW
    NCH = BPW // CH
    NSUP = BPW // SUPER
    assert B % (NW * SUPER) == 0 and NSUP % 2 == 0

    mesh = plsc.VectorSubcoreMesh(core_axis_name="c", subcore_axis_name="s")

    @functools.partial(
        pl.kernel,
        out_type=jax.ShapeDtypeStruct((B, D), jnp.float32),
        mesh=mesh,
        scratch_types=[
            pltpu.VMEM((NCH, CH), jnp.int32),
            pltpu.VMEM((SUPER, D), jnp.float32),
            pltpu.VMEM((SUPER, D), jnp.float32),
            pltpu.SemaphoreType.DMA,
            pltpu.SemaphoreType.DMA,
        ],
        compiler_params=pltpu.CompilerParams(use_tc_tiling_on_sc=False),
    )
    def emb(idx_hbm, table_hbm, out_hbm, idx_v, buf0, buf1, sem0, sem1):
        wid = lax.axis_index("s") * NC + lax.axis_index("c")
        base = wid * BPW
        pltpu.sync_copy(idx_hbm.at[pl.ds(wid * NCH, NCH)], idx_v)
        bufs = (buf0, buf1)
        sems = (sem0, sem1)

        def fire(g, b):
            for j in range(GPS):
                pltpu.make_async_copy(
                    table_hbm.at[idx_v.at[g * GPS + j]],
                    bufs[b].at[pl.ds(j * CH, CH)],
                    sems[b],
                ).start()

        def drain(g, b):
            for j in range(GPS):
                pltpu.make_async_copy(
                    table_hbm.at[idx_v.at[g * GPS + j]],
                    bufs[b].at[pl.ds(j * CH, CH)],
                    sems[b],
                ).wait()

        fire(0, 0)

        @pl.loop(0, NSUP, step=2)
        def _super(g0):
            for b in range(2):
                g = g0 + b

                @pl.when(g + 1 < NSUP)
                def _():
                    fire(g + 1, 1 - b)

                drain(g, b)
                pltpu.sync_copy(
                    bufs[b], out_hbm.at[pl.ds(base + g * SUPER, SUPER)]
                )

    return emb


def kernel(x, table):
    S, T = x.shape
    B = S * T
    idx = x.reshape(B // CH, CH).astype(jnp.int32)
    out = _build(B)(idx, table)
    return out.reshape(S, T, D)
